# Initial kernel scaffold; baseline (speedup 1.0000x reference)
import jax
import jax.numpy as jnp
from jax.experimental import pallas as pl


def kernel(x, y0, y1, z):
    raise NotImplementedError("write your pallas kernel here")


# speedup vs baseline: 8.1449x; 8.1449x over previous
import jax
import jax.numpy as jnp
from jax import lax
from jax.experimental import pallas as pl
from jax.experimental.pallas import tpu as pltpu
from jax.experimental.pallas import tpu_sc as plsc

M, D, B = 100000, 64, 16384
L = 16
NG = D // L
NC = 2
NS = 16
GPC = NG // NC

UPD_ROWS = B // NS
SUB = 256
NSUB = UPD_ROWS // SUB
CHUNK = 128
RPC = CHUNK // L
NCHUNK = SUB * L // CHUNK
SEG = M * L // NS
OUT_ROWS = M // NS
RCH = 125
NRCH = OUT_ROWS // RCH
ZCH = 5000
NZ = SEG // ZCH


def _body(x_hbm, y0_hbm, y1_hbm, z_hbm, out0_hbm, out1_hbm,
          acc, y_v, z_v, idx_v, val_v, zero_v, sbuf, xbuf, obuf,
          sem0, sem1, sem2, sem3):
    c = lax.axis_index("c")
    s = lax.axis_index("s")
    iota = lax.iota(jnp.int32, L)
    sems = (sem0, sem1, sem2, sem3)
    seg0 = pl.multiple_of(s * SEG, 8)

    @pl.loop(0, ZCH // L)
    def _(i):
        zero_v[pl.ds(pl.multiple_of(i * L, L), L)] = jnp.zeros((L,), jnp.float32)

    for y_hbm, out_hbm in ((y0_hbm, out0_hbm), (y1_hbm, out1_hbm)):
        for gg in range(GPC):
            g = c * GPC + gg
            col0 = pl.multiple_of(g * L, L)

            @pl.loop(0, NZ)
            def _(k):
                off = pl.multiple_of(seg0 + k * ZCH, 8)
                pltpu.sync_copy(zero_v, acc.at[pl.ds(off, ZCH)])

            plsc.subcore_barrier()

            @pl.loop(0, NSUB)
            def _(sb):
                r0 = s * UPD_ROWS + sb * SUB
                pltpu.sync_copy(
                    y_hbm.at[pl.ds(r0, SUB), pl.ds(col0, L)], y_v)
                pltpu.sync_copy(
                    z_hbm.at[pl.ds(r0, SUB), pl.ds(col0, L)], z_v)

                @pl.loop(0, NCHUNK)
                def _(cc):
                    for u in range(RPC):
                        r = cc * RPC + u
                        idx_v[cc, pl.ds(u * L, L)] = y_v[r, :] * L + iota
                        val_v[cc, pl.ds(u * L, L)] = z_v[r, :]

                @pl.loop(0, NCHUNK)
                def _(cc):
                    pltpu.async_copy(val_v.at[cc],
                                     acc.at[idx_v.at[cc]],
                                     sems[0], add=True).wait()

            plsc.subcore_barrier()

            @pl.loop(0, NRCH)
            def _(k):
                row0 = s * OUT_ROWS + k * RCH
                aoff = pl.multiple_of(seg0 + k * RCH * L, 8)
                pltpu.sync_copy(acc.at[pl.ds(aoff, RCH * L)], sbuf)
                pltpu.sync_copy(
                    x_hbm.at[pl.ds(row0, RCH), pl.ds(col0, L)], xbuf)

                @pl.loop(0, RCH // 5)
                def _(rr):
                    for u in range(5):
                        r = rr * 5 + u
                        obuf[r, :] = (
                            sbuf[pl.ds(pl.multiple_of(r * L, L), L)]
                            + xbuf[r, :])

                pltpu.sync_copy(
                    obuf, out_hbm.at[pl.ds(row0, RCH), pl.ds(col0, L)])


_sc_call = pl.kernel(
    _body,
    out_type=(
        jax.ShapeDtypeStruct((M, D), jnp.float32),
        jax.ShapeDtypeStruct((M, D), jnp.float32),
    ),
    mesh=plsc.VectorSubcoreMesh(core_axis_name="c", subcore_axis_name="s"),
    compiler_params=pltpu.CompilerParams(use_tc_tiling_on_sc=False),
    scratch_types=[
        pltpu.VMEM_SHARED((M * L,), jnp.float32),
        pltpu.VMEM((SUB, L), jnp.int32),
        pltpu.VMEM((SUB, L), jnp.float32),
        pltpu.VMEM((NCHUNK, CHUNK), jnp.int32),
        pltpu.VMEM((NCHUNK, CHUNK), jnp.float32),
        pltpu.VMEM((ZCH,), jnp.float32),
        pltpu.VMEM((RCH * L,), jnp.float32),
        pltpu.VMEM((RCH, L), jnp.float32),
        pltpu.VMEM((RCH, L), jnp.float32),
        pltpu.SemaphoreType.DMA,
        pltpu.SemaphoreType.DMA,
        pltpu.SemaphoreType.DMA,
        pltpu.SemaphoreType.DMA,
    ],
)


def kernel(x, y0, y1, z):
    return _sc_call(x, y0, y1, z)


# speedup vs baseline: 8.6214x; 1.0585x over previous
import jax
import jax.numpy as jnp
from jax import lax
from jax.experimental import pallas as pl
from jax.experimental.pallas import tpu as pltpu
from jax.experimental.pallas import tpu_sc as plsc

M, D, B = 100000, 64, 16384
L = 16
NG = D // L
NC = 2
NS = 16
GPC = NG // NC

UPD_ROWS = B // NS
SUB = 256
NSUB = UPD_ROWS // SUB
CHUNK = 128
RPC = CHUNK // L
NCHUNK = SUB * L // CHUNK
SEG = M * L // NS
OUT_ROWS = M // NS
RCH = 125
NRCH = OUT_ROWS // RCH
ZCH = 5000
NZ = SEG // ZCH


def _body(x_hbm, y0_hbm, y1_hbm, z_hbm, out0_hbm, out1_hbm,
          acc, y_v, z_v, idx_v, val_v, zero_v, sbuf, xbuf, obuf,
          sem0, sem1, sem2, sem3):
    c = lax.axis_index("c")
    s = lax.axis_index("s")
    iota = lax.iota(jnp.int32, L)
    sems = (sem0, sem1, sem2, sem3)
    seg0 = pl.multiple_of(s * SEG, 8)

    @pl.loop(0, ZCH // L)
    def _(i):
        zero_v[pl.ds(pl.multiple_of(i * L, L), L)] = jnp.zeros((L,), jnp.float32)

    for y_hbm, out_hbm in ((y0_hbm, out0_hbm), (y1_hbm, out1_hbm)):
        for gg in range(GPC):
            g = c * GPC + gg
            col0 = pl.multiple_of(g * L, L)

            @pl.loop(0, NZ)
            def _(k):
                off = pl.multiple_of(seg0 + k * ZCH, 8)
                pltpu.sync_copy(zero_v, acc.at[pl.ds(off, ZCH)])

            plsc.subcore_barrier()

            @pl.loop(0, NSUB)
            def _(sb):
                r0 = s * UPD_ROWS + sb * SUB
                pltpu.sync_copy(
                    y_hbm.at[pl.ds(r0, SUB), pl.ds(col0, L)], y_v)
                pltpu.sync_copy(
                    z_hbm.at[pl.ds(r0, SUB), pl.ds(col0, L)], z_v)

                @pl.loop(0, SUB // RPC)
                def _(cc):
                    for u in range(RPC):
                        r = cc * RPC + u
                        off = pl.multiple_of(r * L, L)
                        idx_v[pl.ds(off, L)] = y_v[r, :] * L + iota
                        val_v[pl.ds(off, L)] = z_v[r, :]

                pltpu.async_copy(val_v, acc.at[idx_v],
                                 sems[0], add=True).wait()

            plsc.subcore_barrier()

            @pl.loop(0, NRCH)
            def _(k):
                row0 = s * OUT_ROWS + k * RCH
                aoff = pl.multiple_of(seg0 + k * RCH * L, 8)
                pltpu.sync_copy(acc.at[pl.ds(aoff, RCH * L)], sbuf)
                pltpu.sync_copy(
                    x_hbm.at[pl.ds(row0, RCH), pl.ds(col0, L)], xbuf)

                @pl.loop(0, RCH // 5)
                def _(rr):
                    for u in range(5):
                        r = rr * 5 + u
                        obuf[r, :] = (
                            sbuf[pl.ds(pl.multiple_of(r * L, L), L)]
                            + xbuf[r, :])

                pltpu.sync_copy(
                    obuf, out_hbm.at[pl.ds(row0, RCH), pl.ds(col0, L)])


_sc_call = pl.kernel(
    _body,
    out_type=(
        jax.ShapeDtypeStruct((M, D), jnp.float32),
        jax.ShapeDtypeStruct((M, D), jnp.float32),
    ),
    mesh=plsc.VectorSubcoreMesh(core_axis_name="c", subcore_axis_name="s"),
    compiler_params=pltpu.CompilerParams(use_tc_tiling_on_sc=False),
    scratch_types=[
        pltpu.VMEM_SHARED((M * L,), jnp.float32),
        pltpu.VMEM((SUB, L), jnp.int32),
        pltpu.VMEM((SUB, L), jnp.float32),
        pltpu.VMEM((SUB * L,), jnp.int32),
        pltpu.VMEM((SUB * L,), jnp.float32),
        pltpu.VMEM((ZCH,), jnp.float32),
        pltpu.VMEM((RCH * L,), jnp.float32),
        pltpu.VMEM((RCH, L), jnp.float32),
        pltpu.VMEM((RCH, L), jnp.float32),
        pltpu.SemaphoreType.DMA,
        pltpu.SemaphoreType.DMA,
        pltpu.SemaphoreType.DMA,
        pltpu.SemaphoreType.DMA,
    ],
)


def kernel(x, y0, y1, z):
    return _sc_call(x, y0, y1, z)
